# baseline (speedup 1.0000x reference)
import functools

import jax
import jax.numpy as jnp
from jax.experimental import pallas as pl

B, S, D, E, K = 4, 2048, 2880, 32, 4
T = 256


def _moe_block(hid_ref, w_ref, b_ref, eo_ref, out_ref):
    scores = jax.lax.dot_general(
        hid_ref[...], w_ref[...],
        dimension_numbers=(((1,), (1,)), ((), ())),
        preferred_element_type=jnp.float32,
    ) + b_ref[...]

    idx = jax.lax.broadcasted_iota(jnp.int32, scores.shape, 1)
    s = scores
    tops = []
    for _ in range(K):
        m = jnp.max(s, axis=1, keepdims=True)
        tops.append(m)
        first = jnp.min(jnp.where(s == m, idx, E), axis=1, keepdims=True)
        s = jnp.where(idx == first, -jnp.inf, s)

    exps = [jnp.exp(t - tops[0]) for t in tops]
    denom = exps[0]
    for e_ in exps[1:]:
        denom = denom + e_
    inv = 1.0 / denom

    acc = (exps[0] * inv) * eo_ref[:, 0, :]
    for k in range(1, K):
        acc = acc + (exps[k] * inv) * eo_ref[:, k, :]
    out_ref[...] = acc


@jax.jit
def kernel(hidden_states, router_weight, router_bias, expert_outputs):
    N = B * S
    hid = hidden_states.reshape(N, D)
    eo = expert_outputs.reshape(N, K, D)
    bias = router_bias.reshape(1, E)

    out = pl.pallas_call(
        _moe_block,
        grid=(N // T,),
        in_specs=[
            pl.BlockSpec((T, D), lambda i: (i, 0)),
            pl.BlockSpec((E, D), lambda i: (0, 0)),
            pl.BlockSpec((1, E), lambda i: (0, 0)),
            pl.BlockSpec((T, K, D), lambda i: (i, 0, 0)),
        ],
        out_specs=pl.BlockSpec((T, D), lambda i: (i, 0)),
        out_shape=jax.ShapeDtypeStruct((N, D), jnp.float32),
    )(hid, router_weight, bias, eo)
    return out.reshape(B, S, D)
